# baseline (speedup 1.0000x reference)
import functools

import jax
import jax.numpy as jnp
from jax import lax
from jax.experimental import pallas as pl
from jax.experimental.pallas import tpu as pltpu
from jax.experimental.pallas import tpu_sc as plsc

_NU = 30000
_NI = 20000
_NN = _NU + _NI
_D = 64
_H = 32
_NE = 800000

_NTILES = 16
_ROWS_PT = 3128
_ACC_N = _ROWS_PT * _NTILES
_HC = 256
_NSLOT = 196
_EPT = _HC * (_NSLOT + 4)
_EP = _EPT * _NTILES


def _fuse_body(tf_ref, imf_ref, ie_ref, tw_ref, iw_ref, w_ref, out_ref):
    t = jnp.dot(tf_ref[...], tw_ref[...], preferred_element_type=jnp.float32)
    t = jnp.maximum(t, 0.0)
    tn = jnp.sqrt(jnp.sum(t * t, axis=1, keepdims=True))
    t = t / jnp.maximum(tn, 1e-12)
    im = jnp.dot(imf_ref[...], iw_ref[...], preferred_element_type=jnp.float32)
    im = jnp.maximum(im, 0.0)
    imn = jnp.sqrt(jnp.sum(im * im, axis=1, keepdims=True))
    im = im / jnp.maximum(imn, 1e-12)
    out_ref[...] = ie_ref[...] + w_ref[0] * t + w_ref[1] * im


def _fuse(item_emb, text_feats, image_feats, text_W, image_W, wt, wi):
    blk = 2000
    grid = _NI // blk
    w = jnp.stack([wt, wi]).astype(jnp.float32)
    return pl.pallas_call(
        _fuse_body,
        grid=(grid,),
        in_specs=[
            pl.BlockSpec((blk, 384), lambda i: (i, 0)),
            pl.BlockSpec((blk, 512), lambda i: (i, 0)),
            pl.BlockSpec((blk, _D), lambda i: (i, 0)),
            pl.BlockSpec((384, _D), lambda i: (0, 0)),
            pl.BlockSpec((512, _D), lambda i: (0, 0)),
            pl.BlockSpec(memory_space=pltpu.SMEM),
        ],
        out_specs=pl.BlockSpec((blk, _D), lambda i: (i, 0)),
        out_shape=jax.ShapeDtypeStruct((_NI, _D), jnp.float32),
    )(text_feats, image_feats, item_emb, text_W, image_W, w)


def _prop_body(user_emb, fused, cols1d, rows1d, vals1d,
               ego0, e1, e2, final2d,
               acc, colsv, valsv, rowsv, g, zbuf, lsem, gsem):
    cid = lax.axis_index("c")
    sid = lax.axis_index("s")

    zero16 = jnp.zeros((16,), jnp.float32)

    def zb_body(r, _):
        zbuf[r, pl.ds(0, 16)] = zero16
        zbuf[r, pl.ds(16, 16)] = zero16
        return 0

    lax.fori_loop(0, 136, zb_body, 0, unroll=4)

    row0 = sid * _ROWS_PT

    def zero_acc():
        def za_body(j, _):
            pltpu.sync_copy(zbuf, acc.at[pl.ds(row0 + j * 136, 136)])
            return 0

        lax.fori_loop(0, 23, za_body, 0)

    zero_acc()

    u0 = sid * 2000
    i0 = sid * 1248

    @pl.when(sid < 15)
    def _():
        pltpu.sync_copy(user_emb.at[pl.ds(u0, 2000), pl.ds(0, _H)],
                        ego0.at[pl.ds(u0, 2000)])
        pltpu.sync_copy(user_emb.at[pl.ds(u0, 2000), pl.ds(_H, _H)],
                        ego0.at[pl.ds(_ACC_N + u0, 2000)])

    pltpu.sync_copy(fused.at[pl.ds(i0, 1248), pl.ds(0, _H)],
                    ego0.at[pl.ds(_NU + i0, 1248)])
    pltpu.sync_copy(fused.at[pl.ds(i0, 1248), pl.ds(_H, _H)],
                    ego0.at[pl.ds(_ACC_N + _NU + i0, 1248)])

    @pl.when(sid == 15)
    def _():
        pltpu.sync_copy(fused.at[pl.ds(19968, 32), pl.ds(0, _H)],
                        ego0.at[pl.ds(_NU + 19968, 32)])
        pltpu.sync_copy(fused.at[pl.ds(19968, 32), pl.ds(_H, _H)],
                        ego0.at[pl.ds(_ACC_N + _NU + 19968, 32)])

    plsc.subcore_barrier()

    coff16 = jnp.full((16,), cid * _ACC_N, jnp.int32)
    e0t = sid * (_NSLOT * _HC)

    def lin_fire(h, b):
        off = e0t + h * _HC
        pltpu.async_copy(cols1d.at[pl.ds(off, _HC)], colsv[b], lsem[b])
        pltpu.async_copy(vals1d.at[pl.ds(off, _HC)], valsv[b], lsem[b])
        pltpu.async_copy(rows1d.at[pl.ds(off, 128)], rowsv[2 * b], lsem[b])
        pltpu.async_copy(rows1d.at[pl.ds(off + 128, 128)], rowsv[2 * b + 1],
                         lsem[b])

    def lin_drain(b):
        off = e0t
        pltpu.make_async_copy(cols1d.at[pl.ds(off, _HC)], colsv[b],
                              lsem[b]).wait()
        pltpu.make_async_copy(vals1d.at[pl.ds(off, _HC)], valsv[b],
                              lsem[b]).wait()
        pltpu.make_async_copy(rows1d.at[pl.ds(off, 128)], rowsv[2 * b],
                              lsem[b]).wait()
        pltpu.make_async_copy(rows1d.at[pl.ds(off, 128)], rowsv[2 * b + 1],
                              lsem[b]).wait()

    def gfire(b, gb, src):
        lin_drain(b)

        def shift_body(k, _):
            colsv[b][pl.ds(k * 16, 16)] = colsv[b][pl.ds(k * 16, 16)] + coff16
            return 0

        lax.fori_loop(0, _HC // 16, shift_body, 0, unroll=8)
        for j in range(2):
            pltpu.async_copy(src.at[colsv[b].at[pl.ds(j * 128, 128)]],
                             g[gb].at[pl.ds(j * 128, 128)], gsem[gb])

    def gwait(b, gb, src):
        for j in range(2):
            pltpu.make_async_copy(src.at[colsv[b].at[pl.ds(j * 128, 128)]],
                                  g[gb].at[pl.ds(j * 128, 128)],
                                  gsem[gb]).wait()

    def consume(b, gb, src):
        gwait(b, gb, src)

        def scale_body(eb, _):
            v16 = valsv[b][pl.ds(eb * 16, 16)]
            for i in range(16):
                vv = lax.broadcast(v16[i], (16,))
                e = eb * 16 + i
                g[gb][e, pl.ds(0, 16)] = g[gb][e, pl.ds(0, 16)] * vv
                g[gb][e, pl.ds(16, 16)] = g[gb][e, pl.ds(16, 16)] * vv
            return 0

        lax.fori_loop(0, _HC // 16, scale_body, 0)
        for j in range(2):
            pltpu.sync_copy(g[gb].at[pl.ds(j * 128, 128)],
                            acc.at[rowsv[2 * b + j]], add=True)

    srcs = (ego0, e1, e2)
    outs = (e1, e2)
    for l in range(3):
        src = srcs[l]
        for b in range(3):
            lin_fire(b, b)
        gfire(0, 0, src)
        lin_fire(3, 3)

        def loop_body(c, _, src=src):
            base = 4 * c + 1
            for i, b in enumerate((1, 2, 3, 0)):
                k = base + i
                gfire(b, (1 + i) % 2, src)
                consume((b - 1) % 4, i % 2, src)
                lin_fire(k + 3, (b - 1) % 4)
            return 0

        lax.fori_loop(0, _NSLOT // 4, loop_body, 0)
        gwait(0, 0, src)
        for b in (1, 2, 3):
            lin_drain(b)

        plsc.subcore_barrier()
        if l < 2:
            pltpu.sync_copy(acc.at[pl.ds(row0, _ROWS_PT)],
                            outs[l].at[pl.ds(cid * _ACC_N + row0, _ROWS_PT)])
            zero_acc()
            plsc.subcore_barrier()

    half = cid * _ACC_N

    def mean_chunk(r0, n):
        pltpu.async_copy(ego0.at[pl.ds(half + r0, n)],
                         g[0].at[pl.ds(0, n)], lsem[0])
        pltpu.async_copy(e1.at[pl.ds(half + r0, n)],
                         g[0].at[pl.ds(128, n)], lsem[1])
        pltpu.async_copy(e2.at[pl.ds(half + r0, n)],
                         g[1].at[pl.ds(0, n)], lsem[2])
        pltpu.async_copy(acc.at[pl.ds(r0, n)],
                         g[1].at[pl.ds(128, n)], lsem[3])
        pltpu.make_async_copy(ego0.at[pl.ds(half + r0, n)],
                              g[0].at[pl.ds(0, n)], lsem[0]).wait()
        pltpu.make_async_copy(e1.at[pl.ds(half + r0, n)],
                              g[0].at[pl.ds(128, n)], lsem[1]).wait()
        pltpu.make_async_copy(e2.at[pl.ds(half + r0, n)],
                              g[1].at[pl.ds(0, n)], lsem[2]).wait()
        pltpu.make_async_copy(acc.at[pl.ds(r0, n)],
                              g[1].at[pl.ds(128, n)], lsem[3]).wait()

        def mean_body(r, _):
            for h2 in range(2):
                d = pl.ds(h2 * 16, 16)
                s = (g[0][r, d] + g[0][128 + r, d]
                     + g[1][r, d] + g[1][128 + r, d])
                g[0][r, d] = s * 0.25
            return 0

        lax.fori_loop(0, n, mean_body, 0, unroll=4)
        pltpu.sync_copy(g[0].at[pl.ds(0, n)],
                        final2d.at[pl.ds(r0, n), pl.ds(cid * _H, _H)])

    def mch_body(ch, _):
        mean_chunk(row0 + ch * 128, 128)
        return 0

    lax.fori_loop(0, 24, mch_body, 0)
    mean_chunk(row0 + 24 * 128, _ROWS_PT - 24 * 128)


def _prop(user_emb, fused, cols1d, rows1d, vals1d):
    mesh = plsc.VectorSubcoreMesh(core_axis_name="c", subcore_axis_name="s")
    f = functools.partial(
        pl.kernel,
        out_type=(
            jax.ShapeDtypeStruct((2 * _ACC_N, _H), jnp.float32),
            jax.ShapeDtypeStruct((2 * _ACC_N, _H), jnp.float32),
            jax.ShapeDtypeStruct((2 * _ACC_N, _H), jnp.float32),
            jax.ShapeDtypeStruct((_ACC_N, _D), jnp.float32),
        ),
        mesh=mesh,
        compiler_params=pltpu.CompilerParams(use_tc_tiling_on_sc=False),
        scratch_types=[
            pltpu.VMEM_SHARED((_ACC_N, _H), jnp.float32),
            [pltpu.VMEM((_HC,), jnp.int32)] * 4,
            [pltpu.VMEM((_HC,), jnp.float32)] * 4,
            [pltpu.VMEM((128,), jnp.int32)] * 8,
            [pltpu.VMEM((_HC, _H), jnp.float32)] * 2,
            pltpu.VMEM((136, _H), jnp.float32),
            [pltpu.SemaphoreType.DMA] * 4,
            [pltpu.SemaphoreType.DMA] * 2,
        ],
    )(_prop_body)
    return f(user_emb, fused, cols1d, rows1d, vals1d)


def kernel(user_emb, item_emb, text_feats, image_feats, text_W, image_W,
           weight_text, weight_image, adj_indices, adj_values):
    fused = _fuse(item_emb, text_feats, image_feats, text_W, image_W,
                  weight_text, weight_image)
    need = 15 * (_NSLOT * _HC) + _EPT
    pad = need - _NE
    rc = jnp.concatenate(
        [adj_indices.astype(jnp.int32), jnp.zeros((2, pad), jnp.int32)],
        axis=1)
    vals1d = jnp.concatenate(
        [adj_values.astype(jnp.float32), jnp.zeros((pad,), jnp.float32)])
    ego0, e1, e2, final2d = _prop(user_emb, fused, rc[1], rc[0], vals1d)
    return final2d[:_NU], final2d[_NU:_NN]


# speedup vs baseline: 1.7638x; 1.7638x over previous
import functools

import jax
import jax.numpy as jnp
from jax import lax
from jax.experimental import pallas as pl
from jax.experimental.pallas import tpu as pltpu
from jax.experimental.pallas import tpu_sc as plsc

_NU = 30000
_NI = 20000
_NN = _NU + _NI
_D = 64
_H = 32
_NE = 800000

_NTILES = 16
_ROWS_PT = 3128
_ACC_N = _ROWS_PT * _NTILES
_HC = 256
_NSLOT = 196
_EPT = _HC * (_NSLOT + 4)
_EP = _EPT * _NTILES


def _fuse_body(tf_ref, imf_ref, ie_ref, tw_ref, iw_ref, w_ref, out_ref):
    t = jnp.dot(tf_ref[...], tw_ref[...], preferred_element_type=jnp.float32)
    t = jnp.maximum(t, 0.0)
    tn = jnp.sqrt(jnp.sum(t * t, axis=1, keepdims=True))
    t = t / jnp.maximum(tn, 1e-12)
    im = jnp.dot(imf_ref[...], iw_ref[...], preferred_element_type=jnp.float32)
    im = jnp.maximum(im, 0.0)
    imn = jnp.sqrt(jnp.sum(im * im, axis=1, keepdims=True))
    im = im / jnp.maximum(imn, 1e-12)
    out_ref[...] = ie_ref[...] + w_ref[0] * t + w_ref[1] * im


def _fuse(item_emb, text_feats, image_feats, text_W, image_W, wt, wi):
    blk = 2000
    grid = _NI // blk
    w = jnp.stack([wt, wi]).astype(jnp.float32)
    return pl.pallas_call(
        _fuse_body,
        grid=(grid,),
        in_specs=[
            pl.BlockSpec((blk, 384), lambda i: (i, 0)),
            pl.BlockSpec((blk, 512), lambda i: (i, 0)),
            pl.BlockSpec((blk, _D), lambda i: (i, 0)),
            pl.BlockSpec((384, _D), lambda i: (0, 0)),
            pl.BlockSpec((512, _D), lambda i: (0, 0)),
            pl.BlockSpec(memory_space=pltpu.SMEM),
        ],
        out_specs=pl.BlockSpec((blk, _D), lambda i: (i, 0)),
        out_shape=jax.ShapeDtypeStruct((_NI, _D), jnp.float32),
    )(text_feats, image_feats, item_emb, text_W, image_W, w)


def _prop_body(ego0, cols1d, rows1d, vals1d,
               e1, e2, final2d,
               acc, colsv, valsv, rowsv, g, zbuf, lsem, gsem):
    cid = lax.axis_index("c")
    sid = lax.axis_index("s")

    zero16 = jnp.zeros((16,), jnp.float32)

    def zb_body(r, _):
        zbuf[r, pl.ds(0, 16)] = zero16
        zbuf[r, pl.ds(16, 16)] = zero16
        return 0

    lax.fori_loop(0, 136, zb_body, 0, unroll=4)

    row0 = sid * _ROWS_PT

    def zero_acc():
        def za_body(j, _):
            pltpu.sync_copy(zbuf, acc.at[pl.ds(row0 + j * 136, 136)])
            return 0

        lax.fori_loop(0, 23, za_body, 0)

    zero_acc()

    plsc.subcore_barrier()

    coff16 = jnp.full((16,), cid * _ACC_N, jnp.int32)
    e0t = sid * (_NSLOT * _HC)

    def lin_fire(h, b):
        off = e0t + h * _HC
        pltpu.async_copy(cols1d.at[pl.ds(off, _HC)], colsv[b], lsem[b])
        pltpu.async_copy(vals1d.at[pl.ds(off, _HC)], valsv[b], lsem[b])
        pltpu.async_copy(rows1d.at[pl.ds(off, 128)], rowsv[2 * b], lsem[b])
        pltpu.async_copy(rows1d.at[pl.ds(off + 128, 128)], rowsv[2 * b + 1],
                         lsem[b])

    def lin_drain(b):
        off = e0t
        pltpu.make_async_copy(cols1d.at[pl.ds(off, _HC)], colsv[b],
                              lsem[b]).wait()
        pltpu.make_async_copy(vals1d.at[pl.ds(off, _HC)], valsv[b],
                              lsem[b]).wait()
        pltpu.make_async_copy(rows1d.at[pl.ds(off, 128)], rowsv[2 * b],
                              lsem[b]).wait()
        pltpu.make_async_copy(rows1d.at[pl.ds(off, 128)], rowsv[2 * b + 1],
                              lsem[b]).wait()

    def gfire(b, gb, src):
        lin_drain(b)

        def shift_body(k, _):
            colsv[b][pl.ds(k * 16, 16)] = colsv[b][pl.ds(k * 16, 16)] + coff16
            return 0

        lax.fori_loop(0, _HC // 16, shift_body, 0, unroll=8)
        for j in range(2):
            pltpu.async_copy(src.at[colsv[b].at[pl.ds(j * 128, 128)]],
                             g[gb].at[pl.ds(j * 128, 128)], gsem[gb])

    def gwait(b, gb, src):
        for j in range(2):
            pltpu.make_async_copy(src.at[colsv[b].at[pl.ds(j * 128, 128)]],
                                  g[gb].at[pl.ds(j * 128, 128)],
                                  gsem[gb]).wait()

    def consume(b, gb, src):
        gwait(b, gb, src)

        def scale_body(eb, _):
            v16 = valsv[b][pl.ds(eb * 16, 16)]
            for i in range(16):
                vv = lax.broadcast(v16[i], (16,))
                e = eb * 16 + i
                g[gb][e, pl.ds(0, 16)] = g[gb][e, pl.ds(0, 16)] * vv
                g[gb][e, pl.ds(16, 16)] = g[gb][e, pl.ds(16, 16)] * vv
            return 0

        lax.fori_loop(0, _HC // 16, scale_body, 0)
        for j in range(2):
            pltpu.sync_copy(g[gb].at[pl.ds(j * 128, 128)],
                            acc.at[rowsv[2 * b + j]], add=True)

    srcs = (ego0, e1, e2)
    outs = (e1, e2)
    for l in range(3):
        src = srcs[l]
        for b in range(3):
            lin_fire(b, b)
        gfire(0, 0, src)
        lin_fire(3, 3)

        def loop_body(c, _, src=src):
            base = 4 * c + 1
            for i, b in enumerate((1, 2, 3, 0)):
                k = base + i
                gfire(b, (1 + i) % 2, src)
                consume((b - 1) % 4, i % 2, src)
                lin_fire(k + 3, (b - 1) % 4)
            return 0

        lax.fori_loop(0, _NSLOT // 4, loop_body, 0)
        gwait(0, 0, src)
        for b in (1, 2, 3):
            lin_drain(b)

        plsc.subcore_barrier()
        if l < 2:
            pltpu.sync_copy(acc.at[pl.ds(row0, _ROWS_PT)],
                            outs[l].at[pl.ds(cid * _ACC_N + row0, _ROWS_PT)])
            zero_acc()
            plsc.subcore_barrier()

    half = cid * _ACC_N

    def mean_chunk(r0, n):
        pltpu.async_copy(ego0.at[pl.ds(half + r0, n)],
                         g[0].at[pl.ds(0, n)], lsem[0])
        pltpu.async_copy(e1.at[pl.ds(half + r0, n)],
                         g[0].at[pl.ds(128, n)], lsem[1])
        pltpu.async_copy(e2.at[pl.ds(half + r0, n)],
                         g[1].at[pl.ds(0, n)], lsem[2])
        pltpu.async_copy(acc.at[pl.ds(r0, n)],
                         g[1].at[pl.ds(128, n)], lsem[3])
        pltpu.make_async_copy(ego0.at[pl.ds(half + r0, n)],
                              g[0].at[pl.ds(0, n)], lsem[0]).wait()
        pltpu.make_async_copy(e1.at[pl.ds(half + r0, n)],
                              g[0].at[pl.ds(128, n)], lsem[1]).wait()
        pltpu.make_async_copy(e2.at[pl.ds(half + r0, n)],
                              g[1].at[pl.ds(0, n)], lsem[2]).wait()
        pltpu.make_async_copy(acc.at[pl.ds(r0, n)],
                              g[1].at[pl.ds(128, n)], lsem[3]).wait()

        def mean_body(r, _):
            for h2 in range(2):
                d = pl.ds(h2 * 16, 16)
                s = (g[0][r, d] + g[0][128 + r, d]
                     + g[1][r, d] + g[1][128 + r, d])
                g[0][r, d] = s * 0.25
            return 0

        lax.fori_loop(0, n, mean_body, 0, unroll=4)
        pltpu.sync_copy(g[0].at[pl.ds(0, n)],
                        final2d.at[pl.ds(r0, n), pl.ds(cid * _H, _H)])

    def mch_body(ch, _):
        mean_chunk(row0 + ch * 128, 128)
        return 0

    lax.fori_loop(0, 24, mch_body, 0)
    mean_chunk(row0 + 24 * 128, _ROWS_PT - 24 * 128)


def _prop(ego_f, cols1d, rows1d, vals1d):
    mesh = plsc.VectorSubcoreMesh(core_axis_name="c", subcore_axis_name="s")
    f = functools.partial(
        pl.kernel,
        out_type=(
            jax.ShapeDtypeStruct((2 * _ACC_N, _H), jnp.float32),
            jax.ShapeDtypeStruct((2 * _ACC_N, _H), jnp.float32),
            jax.ShapeDtypeStruct((_ACC_N, _D), jnp.float32),
        ),
        mesh=mesh,
        compiler_params=pltpu.CompilerParams(use_tc_tiling_on_sc=False),
        scratch_types=[
            pltpu.VMEM_SHARED((_ACC_N, _H), jnp.float32),
            [pltpu.VMEM((_HC,), jnp.int32)] * 4,
            [pltpu.VMEM((_HC,), jnp.float32)] * 4,
            [pltpu.VMEM((128,), jnp.int32)] * 8,
            [pltpu.VMEM((_HC, _H), jnp.float32)] * 2,
            pltpu.VMEM((136, _H), jnp.float32),
            [pltpu.SemaphoreType.DMA] * 4,
            [pltpu.SemaphoreType.DMA] * 2,
        ],
    )(_prop_body)
    return f(ego_f, cols1d, rows1d, vals1d)


def kernel(user_emb, item_emb, text_feats, image_feats, text_W, image_W,
           weight_text, weight_image, adj_indices, adj_values):
    fused = _fuse(item_emb, text_feats, image_feats, text_W, image_W,
                  weight_text, weight_image)
    need = 15 * (_NSLOT * _HC) + _EPT
    pad = need - _NE
    rc = jnp.concatenate(
        [adj_indices.astype(jnp.int32), jnp.zeros((2, pad), jnp.int32)],
        axis=1)
    vals1d = jnp.concatenate(
        [adj_values.astype(jnp.float32), jnp.zeros((pad,), jnp.float32)])
    ego = jnp.concatenate([user_emb, fused], axis=0)
    zpad = jnp.zeros((_ACC_N - _NN, _H), jnp.float32)
    ego_f = jnp.concatenate([ego[:, :_H], zpad, ego[:, _H:], zpad], axis=0)
    e1, e2, final2d = _prop(ego_f, rc[1], rc[0], vals1d)
    return final2d[:_NU], final2d[_NU:_NN]
